# baseline (speedup 1.0000x reference)
import functools

import jax
import jax.numpy as jnp
from jax import lax
from jax.experimental import pallas as pl
from jax.experimental.pallas import tpu as pltpu
from jax.experimental.pallas import tpu_sc as plsc

N = 50000
E = 800000
G = 64
L = 128
NUM_ET = 8
TEXT = 64
HID = 64
OUT = 512

NC = 2
NS = 16
RBLK = 512
NBLK = 98
NPAD = RBLK * NBLK
HALF = NPAD // 2
ACC_ROWS = HALF + 128
TRASH = HALF
CH = 128
NCHK = 392
SUP = 8
EPAD = NS * NCHK * CH
ZSTRIPE = ACC_ROWS // NS
OSTRIPE = HALF // NS

_f32 = jnp.float32
_i32 = jnp.int32


def _aggr_body(h_hbm, idx_hbm, tab_hbm, out_hbm,
               idx_sv, fire0, fire1, psrc, pdst, peid,
               rows0, rows1, bias_v, acc,
               gs0, gs1, ss0, ss1, bsem):
    cid = lax.axis_index("c")
    sid = lax.axis_index("s")
    base = cid * HALF

    def _zrow(i, _):
        for q in range(4):
            rows0[i, pl.ds(q * 16, 16)] = jnp.zeros((16,), _f32)
        return 0
    lax.fori_loop(0, CH, _zrow, 0)

    zoff = sid * ZSTRIPE
    nfull = ZSTRIPE // CH

    def _zcopy(k, _):
        pltpu.sync_copy(rows0, acc.at[pl.ds(zoff + k * CH, CH)])
        return 0
    lax.fori_loop(0, nfull, _zcopy, 0)
    rem = ZSTRIPE - nfull * CH
    pltpu.sync_copy(rows0.at[pl.ds(0, rem)],
                    acc.at[pl.ds(zoff + nfull * CH, rem)])

    plsc.subcore_barrier()

    FIRES = ((fire0, rows0, gs0, ss0), (fire1, rows1, gs1, ss1))

    def _compute(rv, bv):
        @functools.partial(plsc.parallel_loop, 0, CH, unroll=4)
        def _edge(e):
            for q in range(4):
                ds = pl.ds(q * 16, 16)
                rv[e, ds] = jnp.maximum(rv[e, ds] + bv[e, ds], 0.0)

    def _chunk(sk, carry):
        s, k = sk // SUP, sk % SUP
        wp, nf = carry
        @pl.when(k == 0)
        def _():
            pltpu.sync_copy(idx_hbm.at[sid, s], idx_sv)
        for i in range(CH // 16):
            ds = pl.ds(i * 16, 16)
            s16 = idx_sv[k, 0, ds]
            d16 = idx_sv[k, 1, ds]
            e16 = idx_sv[k, 2, ds]
            loc = d16 - base
            m = (loc >= 0) & (loc < HALF)
            lanei = lax.iota(_i32, 16)
            _, sval = plsc.sort_key_val(jnp.where(m, 0, 1), lanei)
            cnt = plsc.all_reduce_population_count(m)[0]
            idx_sv[k, 1, ds] = loc
            idx_sv[k, 2, ds] = jnp.minimum(e16, NUM_ET - 1)
            col = sval + i * 16
            kk = jnp.full((16,), 0, _i32) + k
            sg = plsc.load_gather(idx_sv, [kk, jnp.full((16,), 0, _i32), col])
            lg = plsc.load_gather(idx_sv, [kk, jnp.full((16,), 1, _i32), col])
            eg = plsc.load_gather(idx_sv, [kk, jnp.full((16,), 2, _i32), col])
            keep = lanei < cnt
            pos = wp + lanei
            plsc.store_scatter(psrc, [pos], sg, mask=keep)
            plsc.store_scatter(pdst, [pos], lg, mask=keep)
            plsc.store_scatter(peid, [pos], eg, mask=keep)
            wp = wp + cnt

        fire = wp >= CH
        p = nf & 1
        for ps in (0, 1):
            fv, rv, gs, ss = FIRES[ps]
            qv, qr, qg, qs = FIRES[1 - ps]

            @pl.when(fire & (p == ps))
            def _():
                @pl.when(nf >= 2)
                def _():
                    pltpu.make_async_copy(rv, acc.at[fv.at[1]], ss).wait()
                for i in range(CH // 16):
                    ds = pl.ds(i * 16, 16)
                    fv[0, ds] = psrc[ds]
                    fv[1, ds] = pdst[ds]
                    fv[2, ds] = peid[ds]
                for i in range(CH // 16):
                    ds = pl.ds(i * 16, 16)
                    ds2 = pl.ds(CH + i * 16, 16)
                    psrc[ds] = psrc[ds2]
                    pdst[ds] = pdst[ds2]
                    peid[ds] = peid[ds2]
                pltpu.async_copy(h_hbm.at[fv.at[0]], rv, gs)

                @pl.when(nf >= 1)
                def _():
                    pltpu.make_async_copy(h_hbm.at[qv.at[0]], qr, qg).wait()
                    pltpu.make_async_copy(tab_hbm.at[qv.at[2]], bias_v,
                                          bsem).wait()
                    _compute(qr, bias_v)
                    pltpu.async_copy(qr, acc.at[qv.at[1]], qs, add=True)
                pltpu.async_copy(tab_hbm.at[fv.at[2]], bias_v, bsem)
        wp = jnp.where(fire, wp - CH, wp)
        nf = jnp.where(fire, nf + 1, nf)
        return (wp, nf)
    wp, nf = lax.fori_loop(0, NCHK, _chunk,
                           (jnp.int32(0), jnp.int32(0)))

    for ps in (0, 1):
        fv, rv, gs, ss = FIRES[ps]

        @pl.when((nf >= 1) & (((nf - 1) & 1) == ps))
        def _():
            pltpu.make_async_copy(h_hbm.at[fv.at[0]], rv, gs).wait()
            pltpu.make_async_copy(tab_hbm.at[fv.at[2]], bias_v, bsem).wait()
            _compute(rv, bias_v)
            pltpu.sync_copy(rv, acc.at[fv.at[1]], add=True)

    for ps in (0, 1):
        fv, rv, gs, ss = FIRES[ps]

        @pl.when((nf >= 2) & ((nf & 1) == ps))
        def _():
            pltpu.make_async_copy(rv, acc.at[fv.at[1]], ss).wait()

    for ps in (0, 1):
        fv, rv, gs, ss = FIRES[ps]

        @pl.when((wp > 0) & ((nf & 1) == ps))
        def _():
            for i in range(CH // 16):
                ds = pl.ds(i * 16, 16)
                lane = lax.iota(_i32, 16) + i * 16
                keep = lane < wp
                fv[0, ds] = jnp.where(keep, psrc[ds], 0)
                fv[1, ds] = jnp.where(keep, pdst[ds], TRASH)
                fv[2, ds] = jnp.where(keep, peid[ds], 0)
            pltpu.async_copy(h_hbm.at[fv.at[0]], rv, gs).wait()
            pltpu.async_copy(tab_hbm.at[fv.at[2]], bias_v, bsem).wait()
            _compute(rv, bias_v)
            pltpu.sync_copy(rv, acc.at[fv.at[1]], add=True)

    plsc.subcore_barrier()
    pltpu.sync_copy(acc.at[pl.ds(sid * OSTRIPE, OSTRIPE)],
                    out_hbm.at[pl.ds(base + sid * OSTRIPE, OSTRIPE)])


@functools.lru_cache(maxsize=None)
def _sc_mesh():
    return plsc.VectorSubcoreMesh(core_axis_name="c", subcore_axis_name="s",
                                  num_cores=NC, num_subcores=NS)


@functools.lru_cache(maxsize=None)
def _aggr_kernel():
    return pl.kernel(
        _aggr_body,
        out_type=jax.ShapeDtypeStruct((NPAD, HID), _f32),
        mesh=_sc_mesh(),
        scratch_types=[
            pltpu.VMEM((SUP, 3, CH), _i32),
            pltpu.VMEM((3, CH), _i32),
            pltpu.VMEM((3, CH), _i32),
            pltpu.VMEM((288,), _i32),
            pltpu.VMEM((288,), _i32),
            pltpu.VMEM((288,), _i32),
            pltpu.VMEM((CH, HID), _f32),
            pltpu.VMEM((CH, HID), _f32),
            pltpu.VMEM((CH, HID), _f32),
            pltpu.VMEM_SHARED((ACC_ROWS, HID), _f32),
            pltpu.SemaphoreType.DMA,
            pltpu.SemaphoreType.DMA,
            pltpu.SemaphoreType.DMA,
            pltpu.SemaphoreType.DMA,
            pltpu.SemaphoreType.DMA,
        ],
        compiler_params=pltpu.CompilerParams(use_tc_tiling_on_sc=False,
                                             needs_layout_passes=False),
    )


def _aggr_call(h, idx, tab):
    return _aggr_kernel()(h, idx, tab)


def _text_body(tok_hbm, ids_hbm, mask_hbm, out_hbm,
               ids_v, mask_v, rows_v, res_v, sem):
    cid = lax.axis_index("c")
    sid = lax.axis_index("s")
    w = sid * NC + cid
    g0 = w * 2

    pltpu.sync_copy(ids_hbm.at[pl.ds(g0, 2)], ids_v)
    pltpu.sync_copy(mask_hbm.at[pl.ds(g0, 2)], mask_v)
    for j in range(2):
        pltpu.async_copy(tok_hbm.at[ids_v.at[j]], rows_v, sem).wait()

        def _acc(i, carry):
            a0, a1, a2, a3, ms = carry
            m16 = mask_v[j, pl.ds(i * 16, 16)]
            for lane in range(16):
                m = m16[lane]
                e = i * 16 + lane
                a0 = a0 + rows_v[e, pl.ds(0, 16)] * m
                a1 = a1 + rows_v[e, pl.ds(16, 16)] * m
                a2 = a2 + rows_v[e, pl.ds(32, 16)] * m
                a3 = a3 + rows_v[e, pl.ds(48, 16)] * m
            return (a0, a1, a2, a3, ms + m16)
        z = jnp.zeros((16,), _f32)
        a0, a1, a2, a3, _ = lax.fori_loop(0, L // 16, _acc,
                                          (z, z, z, z, z))
        res_v[j, pl.ds(0, 16)] = a0
        res_v[j, pl.ds(16, 16)] = a1
        res_v[j, pl.ds(32, 16)] = a2
        res_v[j, pl.ds(48, 16)] = a3
    pltpu.sync_copy(res_v, out_hbm.at[pl.ds(g0, 2)])


@functools.lru_cache(maxsize=None)
def _text_kernel():
    return pl.kernel(
        _text_body,
        out_type=jax.ShapeDtypeStruct((G, TEXT), _f32),
        mesh=_sc_mesh(),
        scratch_types=[
            pltpu.VMEM((2, L), _i32),
            pltpu.VMEM((2, L), _f32),
            pltpu.VMEM((L, TEXT), _f32),
            pltpu.VMEM((2, TEXT), _f32),
            pltpu.SemaphoreType.DMA,
        ],
        compiler_params=pltpu.CompilerParams(use_tc_tiling_on_sc=False,
                                             needs_layout_passes=False),
    )


def _text_call(tok_w, ids, mask):
    return _text_kernel()(tok_w, ids, mask)


def _embed_body(x_ref, opw_ref, ew_ref, we0, we1, we2, b0, b1, b2,
                h_ref, t0_ref, t1_ref, t2_ref):
    xb = x_ref[...]
    opi = jnp.clip(xb[:, 0:1].astype(_i32), 0, 63)
    ioh = lax.broadcasted_iota(_i32, (RBLK, 64), 1)
    oh = (opi == ioh).astype(_f32)
    emb = jnp.dot(oh, opw_ref[...], preferred_element_type=_f32)
    h_ref[...] = jnp.concatenate(
        [emb, xb[:, 1:9], jnp.zeros((RBLK, 24), _f32)], axis=1)

    @pl.when(pl.program_id(0) == 0)
    def _():
        ew = ew_ref[...]
        t0_ref[...] = jnp.dot(ew, we0[...], preferred_element_type=_f32) + b0[...]
        t1_ref[...] = jnp.dot(ew, we1[...], preferred_element_type=_f32) + b1[...]
        t2_ref[...] = jnp.dot(ew, we2[...], preferred_element_type=_f32) + b2[...]


def _mlp_body(res):
    def body(h_ref, a_ref, w1, b1, w2, b2, eps_ref, o_ref):
        h = h_ref[...]
        z = (1.0 + eps_ref[0, 0]) * h + a_ref[...]
        a = jnp.maximum(jnp.dot(z, w1[...], preferred_element_type=_f32) + b1[...], 0.0)
        hh = jnp.dot(a, w2[...], preferred_element_type=_f32) + b2[...]
        mu = jnp.mean(hh, axis=-1, keepdims=True)
        var = jnp.mean((hh - mu) ** 2, axis=-1, keepdims=True)
        ln = (hh - mu) / jnp.sqrt(var + 1e-5)
        y = ln + h if res else ln
        o_ref[...] = jnp.where(y >= 0, y, 0.1 * y)
    return body


def _pool_body(h_ref, x_ref, text_ref, mask_ref, wm1, bm1, wm2, bm2,
               o_ref, accA, accB):
    i = pl.program_id(0)

    @pl.when(i == 0)
    def _():
        accA[...] = jnp.zeros_like(accA)
        accB[...] = jnp.zeros_like(accB)

    xb = x_ref[...]
    bfl = xb[:, 9:10].astype(_i32)
    ioh = lax.broadcasted_iota(_i32, (RBLK, G), 1)
    oh = (bfl == ioh).astype(_f32)
    accA[...] += lax.dot_general(oh, h_ref[...], (((0,), (0,)), ((), ())),
                                 preferred_element_type=_f32)
    cols = jnp.concatenate(
        [jnp.ones((RBLK, 1), _f32), xb[:, 5:6], xb[:, 4:5],
         jnp.zeros((RBLK, 5), _f32)], axis=1)
    accB[...] += lax.dot_general(oh, cols, (((0,), (0,)), ((), ())),
                                 preferred_element_type=_f32)

    @pl.when(i == NBLK - 1)
    def _():
        B = accB[...]
        cnt = B[:, 0:1]
        safe = jnp.where(cnt > 0, cnt, 1.0)
        lengths = jnp.maximum(jnp.sum(mask_ref[...], axis=1, keepdims=True), 1.0)
        cat = jnp.concatenate(
            [accA[...], cnt, B[:, 1:2] / safe, B[:, 2:3] / safe,
             text_ref[...] / lengths, jnp.zeros((G, 5), _f32)], axis=1)
        hid = jnp.dot(cat, wm1[...], preferred_element_type=_f32) + bm1[...]
        hid = jnp.where(hid >= 0, hid, 0.1 * hid)
        o_ref[...] = jnp.dot(hid, wm2[...], preferred_element_type=_f32) + bm2[...]


def kernel(x, edge_index, edge_attr, batch, sql_ids, sql_mask, params):
    xp = jnp.zeros((NPAD, 128), _f32)
    xp = xp.at[:N, :9].set(x)
    xp = xp.at[:N, 9].set(batch.astype(_f32))
    xp = xp.at[N:, 9].set(127.0)

    pad_e = EPAD - E
    srcp = jnp.concatenate([edge_index[0].astype(_i32),
                            jnp.zeros((pad_e,), _i32)])
    dstp = jnp.concatenate([edge_index[1].astype(_i32),
                            jnp.full((pad_e,), NPAD, _i32)])
    eidp = jnp.concatenate([edge_attr.astype(_i32), jnp.zeros((pad_e,), _i32)])
    idx = jnp.stack([srcp, dstp, eidp], 0)
    idx = idx.reshape(3, NS, NCHK, CH).transpose(1, 2, 0, 3)
    idx = idx.reshape(NS, NCHK // SUP, SUP, 3, CH)

    convs = params['convs']
    ins = [40, 64, 64]
    wes, bes, w1s, b1s, w2s, b2s, epss = [], [], [], [], [], [], []
    for l, c in enumerate(convs):
        wes.append(jnp.zeros((16, HID), _f32).at[:, :ins[l]].set(c['We']))
        bes.append(jnp.zeros((1, HID), _f32).at[0, :ins[l]].set(c['be']))
        w1s.append(jnp.zeros((HID, HID), _f32).at[:ins[l], :].set(c['W1']))
        b1s.append(c['b1'].reshape(1, HID))
        w2s.append(c['W2'])
        b2s.append(c['b2'].reshape(1, HID))
        epss.append(jnp.asarray(c['eps'], _f32).reshape(1, 1))

    wm1 = jnp.zeros((136, HID), _f32).at[:131, :].set(params['Wm1'])
    bm1 = params['bm1'].reshape(1, HID)
    wm2 = params['Wm2']
    bm2 = params['bm2'].reshape(1, OUT)

    grid = (NBLK,)
    full = lambda s: pl.BlockSpec(s, lambda i: (0, 0))
    h0, t0, t1, t2 = pl.pallas_call(
        _embed_body,
        grid=grid,
        in_specs=[pl.BlockSpec((RBLK, 128), lambda i: (i, 0)),
                  full((64, 32)), full((NUM_ET, 16)),
                  full((16, HID)), full((16, HID)), full((16, HID)),
                  full((1, HID)), full((1, HID)), full((1, HID))],
        out_specs=[pl.BlockSpec((RBLK, HID), lambda i: (i, 0)),
                   full((NUM_ET, HID)), full((NUM_ET, HID)), full((NUM_ET, HID))],
        out_shape=[jax.ShapeDtypeStruct((NPAD, HID), _f32)] +
                  [jax.ShapeDtypeStruct((NUM_ET, HID), _f32)] * 3,
    )(xp, params['op_w'], params['edge_w'],
      wes[0], wes[1], wes[2], bes[0], bes[1], bes[2])

    tabs = [t0, t1, t2]
    h = h0
    for l in range(3):
        aggr = _aggr_call(h, idx, tabs[l])
        h = pl.pallas_call(
            _mlp_body(l > 0),
            grid=grid,
            in_specs=[pl.BlockSpec((RBLK, HID), lambda i: (i, 0)),
                      pl.BlockSpec((RBLK, HID), lambda i: (i, 0)),
                      full((HID, HID)), full((1, HID)),
                      full((HID, HID)), full((1, HID)),
                      full((1, 1))],
            out_specs=pl.BlockSpec((RBLK, HID), lambda i: (i, 0)),
            out_shape=jax.ShapeDtypeStruct((NPAD, HID), _f32),
        )(h, aggr, w1s[l], b1s[l], w2s[l], b2s[l], epss[l])

    text = _text_call(params['tok_w'], sql_ids.astype(_i32), sql_mask)

    out = pl.pallas_call(
        _pool_body,
        grid=grid,
        in_specs=[pl.BlockSpec((RBLK, HID), lambda i: (i, 0)),
                  pl.BlockSpec((RBLK, 128), lambda i: (i, 0)),
                  full((G, TEXT)), full((G, L)), full((136, HID)),
                  full((1, HID)), full((HID, OUT)), full((1, OUT))],
        out_specs=full((G, OUT)),
        out_shape=jax.ShapeDtypeStruct((G, OUT), _f32),
        scratch_shapes=[pltpu.VMEM((G, HID), _f32), pltpu.VMEM((G, 8), _f32)],
    )(h, xp, text, sql_mask.astype(_f32), wm1, bm1, wm2, bm2)
    return out


# speedup vs baseline: 4.1563x; 4.1563x over previous
import functools

import jax
import jax.numpy as jnp
from jax import lax
from jax.experimental import pallas as pl
from jax.experimental.pallas import tpu as pltpu
from jax.experimental.pallas import tpu_sc as plsc

N = 50000
E = 800000
G = 64
L = 128
NUM_ET = 8
TEXT = 64
HID = 64
OUT = 512

NC = 2
NS = 16
RBLK = 512
NBLK = 98
NPAD = RBLK * NBLK
HALF = NPAD // 2
ACC_ROWS = HALF + 128
TRASH = HALF
CH = 128
NCHK = 392
SUP = 8
EPAD = NS * NCHK * CH
ZSTRIPE = ACC_ROWS // NS
OSTRIPE = HALF // NS

_f32 = jnp.float32
_i32 = jnp.int32


def _aggr_body(h_hbm, idx_hbm, tab_hbm, out_hbm,
               idx_sv, fire0, fire1, psrc, pdst, peid,
               rows0, rows1, tab_v, acc,
               gs0, gs1, ss0, ss1, bsem):
    cid = lax.axis_index("c")
    sid = lax.axis_index("s")
    base = cid * HALF

    def _zrow(i, _):
        for q in range(4):
            rows0[i, pl.ds(q * 16, 16)] = jnp.zeros((16,), _f32)
        return 0
    lax.fori_loop(0, CH, _zrow, 0)

    zoff = sid * ZSTRIPE
    nfull = ZSTRIPE // CH

    def _zcopy(k, _):
        pltpu.sync_copy(rows0, acc.at[pl.ds(zoff + k * CH, CH)])
        return 0
    lax.fori_loop(0, nfull, _zcopy, 0)
    rem = ZSTRIPE - nfull * CH
    pltpu.sync_copy(rows0.at[pl.ds(0, rem)],
                    acc.at[pl.ds(zoff + nfull * CH, rem)])

    pltpu.sync_copy(tab_hbm, tab_v)
    plsc.subcore_barrier()

    FIRES = ((fire0, rows0, gs0, ss0), (fire1, rows1, gs1, ss1))

    def _compute(fv, rv):
        def _egroup(i, _):
            t16 = fv[2, pl.ds(i * 16, 16)]
            for lane in range(16):
                t = t16[lane]
                e = i * 16 + lane
                for q in range(4):
                    ds = pl.ds(q * 16, 16)
                    rv[e, ds] = jnp.maximum(rv[e, ds] + tab_v[t, ds], 0.0)
            return 0
        lax.fori_loop(0, CH // 16, _egroup, 0)

    def _chunk(sk, carry):
        s, k = sk // SUP, sk % SUP
        wp, nf = carry
        @pl.when(k == 0)
        def _():
            pltpu.sync_copy(idx_hbm.at[sid, s], idx_sv)
        for i in range(CH // 16):
            ds = pl.ds(i * 16, 16)
            s16 = idx_sv[k, 0, ds]
            d16 = idx_sv[k, 1, ds]
            e16 = idx_sv[k, 2, ds]
            loc = d16 - base
            m = (loc >= 0) & (loc < HALF)
            lanei = lax.iota(_i32, 16)
            _, sval = plsc.sort_key_val(jnp.where(m, 0, 1), lanei)
            cnt = plsc.all_reduce_population_count(m)[0]
            idx_sv[k, 1, ds] = loc
            idx_sv[k, 2, ds] = jnp.minimum(e16, NUM_ET - 1)
            col = sval + i * 16
            kk = jnp.full((16,), 0, _i32) + k
            sg = plsc.load_gather(idx_sv, [kk, jnp.full((16,), 0, _i32), col])
            lg = plsc.load_gather(idx_sv, [kk, jnp.full((16,), 1, _i32), col])
            eg = plsc.load_gather(idx_sv, [kk, jnp.full((16,), 2, _i32), col])
            keep = lanei < cnt
            pos = wp + lanei
            plsc.store_scatter(psrc, [pos], sg, mask=keep)
            plsc.store_scatter(pdst, [pos], lg, mask=keep)
            plsc.store_scatter(peid, [pos], eg, mask=keep)
            wp = wp + cnt

        fire = wp >= CH
        p = nf & 1
        for ps in (0, 1):
            fv, rv, gs, ss = FIRES[ps]
            qv, qr, qg, qs = FIRES[1 - ps]

            @pl.when(fire & (p == ps))
            def _():
                @pl.when(nf >= 2)
                def _():
                    pltpu.make_async_copy(rv, acc.at[fv.at[1]], ss).wait()
                for i in range(CH // 16):
                    ds = pl.ds(i * 16, 16)
                    fv[0, ds] = psrc[ds]
                    fv[1, ds] = pdst[ds]
                    fv[2, ds] = peid[ds]
                for i in range(CH // 16):
                    ds = pl.ds(i * 16, 16)
                    ds2 = pl.ds(CH + i * 16, 16)
                    psrc[ds] = psrc[ds2]
                    pdst[ds] = pdst[ds2]
                    peid[ds] = peid[ds2]
                pltpu.async_copy(h_hbm.at[fv.at[0]], rv, gs)

                @pl.when(nf >= 1)
                def _():
                    pltpu.make_async_copy(h_hbm.at[qv.at[0]], qr, qg).wait()
                    _compute(qv, qr)
                    pltpu.async_copy(qr, acc.at[qv.at[1]], qs, add=True)
        wp = jnp.where(fire, wp - CH, wp)
        nf = jnp.where(fire, nf + 1, nf)
        return (wp, nf)
    wp, nf = lax.fori_loop(0, NCHK, _chunk,
                           (jnp.int32(0), jnp.int32(0)))

    for ps in (0, 1):
        fv, rv, gs, ss = FIRES[ps]

        @pl.when((nf >= 1) & (((nf - 1) & 1) == ps))
        def _():
            pltpu.make_async_copy(h_hbm.at[fv.at[0]], rv, gs).wait()
            _compute(fv, rv)
            pltpu.sync_copy(rv, acc.at[fv.at[1]], add=True)

    for ps in (0, 1):
        fv, rv, gs, ss = FIRES[ps]

        @pl.when((nf >= 2) & ((nf & 1) == ps))
        def _():
            pltpu.make_async_copy(rv, acc.at[fv.at[1]], ss).wait()

    for ps in (0, 1):
        fv, rv, gs, ss = FIRES[ps]

        @pl.when((wp > 0) & ((nf & 1) == ps))
        def _():
            for i in range(CH // 16):
                ds = pl.ds(i * 16, 16)
                lane = lax.iota(_i32, 16) + i * 16
                keep = lane < wp
                fv[0, ds] = jnp.where(keep, psrc[ds], 0)
                fv[1, ds] = jnp.where(keep, pdst[ds], TRASH)
                fv[2, ds] = jnp.where(keep, peid[ds], 0)
            pltpu.async_copy(h_hbm.at[fv.at[0]], rv, gs).wait()
            _compute(fv, rv)
            pltpu.sync_copy(rv, acc.at[fv.at[1]], add=True)

    plsc.subcore_barrier()
    pltpu.sync_copy(acc.at[pl.ds(sid * OSTRIPE, OSTRIPE)],
                    out_hbm.at[pl.ds(base + sid * OSTRIPE, OSTRIPE)])


@functools.lru_cache(maxsize=None)
def _sc_mesh():
    return plsc.VectorSubcoreMesh(core_axis_name="c", subcore_axis_name="s",
                                  num_cores=NC, num_subcores=NS)


@functools.lru_cache(maxsize=None)
def _aggr_kernel():
    return pl.kernel(
        _aggr_body,
        out_type=jax.ShapeDtypeStruct((NPAD, HID), _f32),
        mesh=_sc_mesh(),
        scratch_types=[
            pltpu.VMEM((SUP, 3, CH), _i32),
            pltpu.VMEM((3, CH), _i32),
            pltpu.VMEM((3, CH), _i32),
            pltpu.VMEM((288,), _i32),
            pltpu.VMEM((288,), _i32),
            pltpu.VMEM((288,), _i32),
            pltpu.VMEM((CH, HID), _f32),
            pltpu.VMEM((CH, HID), _f32),
            pltpu.VMEM((NUM_ET, HID), _f32),
            pltpu.VMEM_SHARED((ACC_ROWS, HID), _f32),
            pltpu.SemaphoreType.DMA,
            pltpu.SemaphoreType.DMA,
            pltpu.SemaphoreType.DMA,
            pltpu.SemaphoreType.DMA,
            pltpu.SemaphoreType.DMA,
        ],
        compiler_params=pltpu.CompilerParams(use_tc_tiling_on_sc=False,
                                             needs_layout_passes=False),
    )


def _aggr_call(h, idx, tab):
    return _aggr_kernel()(h, idx, tab)


def _text_body(tok_hbm, ids_hbm, mask_hbm, out_hbm,
               ids_v, mask_v, rows_v, res_v, sem):
    cid = lax.axis_index("c")
    sid = lax.axis_index("s")
    w = sid * NC + cid
    g0 = w * 2

    pltpu.sync_copy(ids_hbm.at[pl.ds(g0, 2)], ids_v)
    pltpu.sync_copy(mask_hbm.at[pl.ds(g0, 2)], mask_v)
    for j in range(2):
        pltpu.async_copy(tok_hbm.at[ids_v.at[j]], rows_v, sem).wait()

        def _acc(i, carry):
            a0, a1, a2, a3, ms = carry
            m16 = mask_v[j, pl.ds(i * 16, 16)]
            for lane in range(16):
                m = m16[lane]
                e = i * 16 + lane
                a0 = a0 + rows_v[e, pl.ds(0, 16)] * m
                a1 = a1 + rows_v[e, pl.ds(16, 16)] * m
                a2 = a2 + rows_v[e, pl.ds(32, 16)] * m
                a3 = a3 + rows_v[e, pl.ds(48, 16)] * m
            return (a0, a1, a2, a3, ms + m16)
        z = jnp.zeros((16,), _f32)
        a0, a1, a2, a3, _ = lax.fori_loop(0, L // 16, _acc,
                                          (z, z, z, z, z))
        res_v[j, pl.ds(0, 16)] = a0
        res_v[j, pl.ds(16, 16)] = a1
        res_v[j, pl.ds(32, 16)] = a2
        res_v[j, pl.ds(48, 16)] = a3
    pltpu.sync_copy(res_v, out_hbm.at[pl.ds(g0, 2)])


@functools.lru_cache(maxsize=None)
def _text_kernel():
    return pl.kernel(
        _text_body,
        out_type=jax.ShapeDtypeStruct((G, TEXT), _f32),
        mesh=_sc_mesh(),
        scratch_types=[
            pltpu.VMEM((2, L), _i32),
            pltpu.VMEM((2, L), _f32),
            pltpu.VMEM((L, TEXT), _f32),
            pltpu.VMEM((2, TEXT), _f32),
            pltpu.SemaphoreType.DMA,
        ],
        compiler_params=pltpu.CompilerParams(use_tc_tiling_on_sc=False,
                                             needs_layout_passes=False),
    )


def _text_call(tok_w, ids, mask):
    return _text_kernel()(tok_w, ids, mask)


def _embed_body(x_ref, opw_ref, ew_ref, we0, we1, we2, b0, b1, b2,
                h_ref, t0_ref, t1_ref, t2_ref):
    xb = x_ref[...]
    opi = jnp.clip(xb[:, 0:1].astype(_i32), 0, 63)
    ioh = lax.broadcasted_iota(_i32, (RBLK, 64), 1)
    oh = (opi == ioh).astype(_f32)
    emb = jnp.dot(oh, opw_ref[...], preferred_element_type=_f32)
    h_ref[...] = jnp.concatenate(
        [emb, xb[:, 1:9], jnp.zeros((RBLK, 24), _f32)], axis=1)

    @pl.when(pl.program_id(0) == 0)
    def _():
        ew = ew_ref[...]
        t0_ref[...] = jnp.dot(ew, we0[...], preferred_element_type=_f32) + b0[...]
        t1_ref[...] = jnp.dot(ew, we1[...], preferred_element_type=_f32) + b1[...]
        t2_ref[...] = jnp.dot(ew, we2[...], preferred_element_type=_f32) + b2[...]


def _mlp_body(res):
    def body(h_ref, a_ref, w1, b1, w2, b2, eps_ref, o_ref):
        h = h_ref[...]
        z = (1.0 + eps_ref[0, 0]) * h + a_ref[...]
        a = jnp.maximum(jnp.dot(z, w1[...], preferred_element_type=_f32) + b1[...], 0.0)
        hh = jnp.dot(a, w2[...], preferred_element_type=_f32) + b2[...]
        mu = jnp.mean(hh, axis=-1, keepdims=True)
        var = jnp.mean((hh - mu) ** 2, axis=-1, keepdims=True)
        ln = (hh - mu) / jnp.sqrt(var + 1e-5)
        y = ln + h if res else ln
        o_ref[...] = jnp.where(y >= 0, y, 0.1 * y)
    return body


def _pool_body(h_ref, x_ref, text_ref, mask_ref, wm1, bm1, wm2, bm2,
               o_ref, accA, accB):
    i = pl.program_id(0)

    @pl.when(i == 0)
    def _():
        accA[...] = jnp.zeros_like(accA)
        accB[...] = jnp.zeros_like(accB)

    xb = x_ref[...]
    bfl = xb[:, 9:10].astype(_i32)
    ioh = lax.broadcasted_iota(_i32, (RBLK, G), 1)
    oh = (bfl == ioh).astype(_f32)
    accA[...] += lax.dot_general(oh, h_ref[...], (((0,), (0,)), ((), ())),
                                 preferred_element_type=_f32)
    cols = jnp.concatenate(
        [jnp.ones((RBLK, 1), _f32), xb[:, 5:6], xb[:, 4:5],
         jnp.zeros((RBLK, 5), _f32)], axis=1)
    accB[...] += lax.dot_general(oh, cols, (((0,), (0,)), ((), ())),
                                 preferred_element_type=_f32)

    @pl.when(i == NBLK - 1)
    def _():
        B = accB[...]
        cnt = B[:, 0:1]
        safe = jnp.where(cnt > 0, cnt, 1.0)
        lengths = jnp.maximum(jnp.sum(mask_ref[...], axis=1, keepdims=True), 1.0)
        cat = jnp.concatenate(
            [accA[...], cnt, B[:, 1:2] / safe, B[:, 2:3] / safe,
             text_ref[...] / lengths, jnp.zeros((G, 5), _f32)], axis=1)
        hid = jnp.dot(cat, wm1[...], preferred_element_type=_f32) + bm1[...]
        hid = jnp.where(hid >= 0, hid, 0.1 * hid)
        o_ref[...] = jnp.dot(hid, wm2[...], preferred_element_type=_f32) + bm2[...]


def kernel(x, edge_index, edge_attr, batch, sql_ids, sql_mask, params):
    xp = jnp.zeros((NPAD, 128), _f32)
    xp = xp.at[:N, :9].set(x)
    xp = xp.at[:N, 9].set(batch.astype(_f32))
    xp = xp.at[N:, 9].set(127.0)

    pad_e = EPAD - E
    srcp = jnp.concatenate([edge_index[0].astype(_i32),
                            jnp.zeros((pad_e,), _i32)])
    dstp = jnp.concatenate([edge_index[1].astype(_i32),
                            jnp.full((pad_e,), NPAD, _i32)])
    eidp = jnp.concatenate([edge_attr.astype(_i32), jnp.zeros((pad_e,), _i32)])
    idx = jnp.stack([srcp, dstp, eidp], 0)
    idx = idx.reshape(3, NS, NCHK, CH).transpose(1, 2, 0, 3)
    idx = idx.reshape(NS, NCHK // SUP, SUP, 3, CH)

    convs = params['convs']
    ins = [40, 64, 64]
    wes, bes, w1s, b1s, w2s, b2s, epss = [], [], [], [], [], [], []
    for l, c in enumerate(convs):
        wes.append(jnp.zeros((16, HID), _f32).at[:, :ins[l]].set(c['We']))
        bes.append(jnp.zeros((1, HID), _f32).at[0, :ins[l]].set(c['be']))
        w1s.append(jnp.zeros((HID, HID), _f32).at[:ins[l], :].set(c['W1']))
        b1s.append(c['b1'].reshape(1, HID))
        w2s.append(c['W2'])
        b2s.append(c['b2'].reshape(1, HID))
        epss.append(jnp.asarray(c['eps'], _f32).reshape(1, 1))

    wm1 = jnp.zeros((136, HID), _f32).at[:131, :].set(params['Wm1'])
    bm1 = params['bm1'].reshape(1, HID)
    wm2 = params['Wm2']
    bm2 = params['bm2'].reshape(1, OUT)

    grid = (NBLK,)
    full = lambda s: pl.BlockSpec(s, lambda i: (0, 0))
    h0, t0, t1, t2 = pl.pallas_call(
        _embed_body,
        grid=grid,
        in_specs=[pl.BlockSpec((RBLK, 128), lambda i: (i, 0)),
                  full((64, 32)), full((NUM_ET, 16)),
                  full((16, HID)), full((16, HID)), full((16, HID)),
                  full((1, HID)), full((1, HID)), full((1, HID))],
        out_specs=[pl.BlockSpec((RBLK, HID), lambda i: (i, 0)),
                   full((NUM_ET, HID)), full((NUM_ET, HID)), full((NUM_ET, HID))],
        out_shape=[jax.ShapeDtypeStruct((NPAD, HID), _f32)] +
                  [jax.ShapeDtypeStruct((NUM_ET, HID), _f32)] * 3,
    )(xp, params['op_w'], params['edge_w'],
      wes[0], wes[1], wes[2], bes[0], bes[1], bes[2])

    tabs = [t0, t1, t2]
    h = h0
    for l in range(3):
        aggr = _aggr_call(h, idx, tabs[l])
        h = pl.pallas_call(
            _mlp_body(l > 0),
            grid=grid,
            in_specs=[pl.BlockSpec((RBLK, HID), lambda i: (i, 0)),
                      pl.BlockSpec((RBLK, HID), lambda i: (i, 0)),
                      full((HID, HID)), full((1, HID)),
                      full((HID, HID)), full((1, HID)),
                      full((1, 1))],
            out_specs=pl.BlockSpec((RBLK, HID), lambda i: (i, 0)),
            out_shape=jax.ShapeDtypeStruct((NPAD, HID), _f32),
        )(h, aggr, w1s[l], b1s[l], w2s[l], b2s[l], epss[l])

    text = _text_call(params['tok_w'], sql_ids.astype(_i32), sql_mask)

    out = pl.pallas_call(
        _pool_body,
        grid=grid,
        in_specs=[pl.BlockSpec((RBLK, HID), lambda i: (i, 0)),
                  pl.BlockSpec((RBLK, 128), lambda i: (i, 0)),
                  full((G, TEXT)), full((G, L)), full((136, HID)),
                  full((1, HID)), full((HID, OUT)), full((1, OUT))],
        out_specs=full((G, OUT)),
        out_shape=jax.ShapeDtypeStruct((G, OUT), _f32),
        scratch_shapes=[pltpu.VMEM((G, HID), _f32), pltpu.VMEM((G, 8), _f32)],
    )(h, xp, text, sql_mask.astype(_f32), wm1, bm1, wm2, bm2)
    return out


# speedup vs baseline: 4.3023x; 1.0351x over previous
import functools

import jax
import jax.numpy as jnp
from jax import lax
from jax.experimental import pallas as pl
from jax.experimental.pallas import tpu as pltpu
from jax.experimental.pallas import tpu_sc as plsc

N = 50000
E = 800000
G = 64
L = 128
NUM_ET = 8
TEXT = 64
HID = 64
OUT = 512

NC = 2
NS = 16
RBLK = 512
NBLK = 98
NPAD = RBLK * NBLK
HALF = NPAD // 2
ACC_ROWS = HALF + 128
TRASH = HALF
CH = 128
NCHK = 392
SUP = 8
EPAD = NS * NCHK * CH
ZSTRIPE = ACC_ROWS // NS
OSTRIPE = HALF // NS

_f32 = jnp.float32
_i32 = jnp.int32


def _aggr_body(h_hbm, idx_hbm, tab_hbm, out_hbm,
               idx_sv, fire0, fire1, psrc, pdst, peid,
               rows0, rows1, tab_v, acc,
               gs0, gs1, ss0, ss1, bsem):
    cid = lax.axis_index("c")
    sid = lax.axis_index("s")
    base = cid * HALF

    def _zrow(i, _):
        for q in range(4):
            rows0[i, pl.ds(q * 16, 16)] = jnp.zeros((16,), _f32)
        return 0
    lax.fori_loop(0, CH, _zrow, 0)

    zoff = sid * ZSTRIPE
    nfull = ZSTRIPE // CH

    def _zcopy(k, _):
        pltpu.sync_copy(rows0, acc.at[pl.ds(zoff + k * CH, CH)])
        return 0
    lax.fori_loop(0, nfull, _zcopy, 0)
    rem = ZSTRIPE - nfull * CH
    pltpu.sync_copy(rows0.at[pl.ds(0, rem)],
                    acc.at[pl.ds(zoff + nfull * CH, rem)])

    pltpu.sync_copy(tab_hbm, tab_v)
    plsc.subcore_barrier()

    FIRES = ((fire0, rows0, gs0, ss0), (fire1, rows1, gs1, ss1))

    def _compute(fv, rv):
        def _egroup(i, _):
            t16 = fv[2, pl.ds(i * 16, 16)]
            for lane in range(16):
                t = t16[lane]
                e = i * 16 + lane
                for q in range(4):
                    ds = pl.ds(q * 16, 16)
                    rv[e, ds] = jnp.maximum(rv[e, ds] + tab_v[t, ds], 0.0)
            return 0
        lax.fori_loop(0, CH // 16, _egroup, 0, unroll=2)

    def _chunk(sk, carry):
        s, k = sk // SUP, sk % SUP
        wp, nf = carry
        @pl.when(k == 0)
        def _():
            pltpu.sync_copy(idx_hbm.at[sid, s], idx_sv)
        for i in range(CH // 16):
            ds = pl.ds(i * 16, 16)
            s16 = idx_sv[k, 0, ds]
            d16 = idx_sv[k, 1, ds]
            e16 = idx_sv[k, 2, ds]
            loc = d16 - base
            m = (loc >= 0) & (loc < HALF)
            lanei = lax.iota(_i32, 16)
            _, sval = plsc.sort_key_val(jnp.where(m, 0, 1), lanei)
            cnt = plsc.all_reduce_population_count(m)[0]
            idx_sv[k, 1, ds] = loc
            idx_sv[k, 2, ds] = jnp.minimum(e16, NUM_ET - 1)
            col = sval + i * 16
            kk = jnp.full((16,), 0, _i32) + k
            sg = plsc.load_gather(idx_sv, [kk, jnp.full((16,), 0, _i32), col])
            lg = plsc.load_gather(idx_sv, [kk, jnp.full((16,), 1, _i32), col])
            eg = plsc.load_gather(idx_sv, [kk, jnp.full((16,), 2, _i32), col])
            keep = lanei < cnt
            pos = wp + lanei
            plsc.store_scatter(psrc, [pos], sg, mask=keep)
            plsc.store_scatter(pdst, [pos], lg, mask=keep)
            plsc.store_scatter(peid, [pos], eg, mask=keep)
            wp = wp + cnt

        fire = wp >= CH
        p = nf & 1
        for ps in (0, 1):
            fv, rv, gs, ss = FIRES[ps]
            qv, qr, qg, qs = FIRES[1 - ps]

            @pl.when(fire & (p == ps))
            def _():
                @pl.when(nf >= 2)
                def _():
                    pltpu.make_async_copy(rv, acc.at[fv.at[1]], ss).wait()
                for i in range(CH // 16):
                    ds = pl.ds(i * 16, 16)
                    fv[0, ds] = psrc[ds]
                    fv[1, ds] = pdst[ds]
                    fv[2, ds] = peid[ds]
                for i in range(CH // 16):
                    ds = pl.ds(i * 16, 16)
                    ds2 = pl.ds(CH + i * 16, 16)
                    psrc[ds] = psrc[ds2]
                    pdst[ds] = pdst[ds2]
                    peid[ds] = peid[ds2]
                pltpu.async_copy(h_hbm.at[fv.at[0]], rv, gs)

                @pl.when(nf >= 1)
                def _():
                    pltpu.make_async_copy(h_hbm.at[qv.at[0]], qr, qg).wait()
                    _compute(qv, qr)
                    pltpu.async_copy(qr, acc.at[qv.at[1]], qs, add=True)
        wp = jnp.where(fire, wp - CH, wp)
        nf = jnp.where(fire, nf + 1, nf)
        return (wp, nf)
    wp, nf = lax.fori_loop(0, NCHK, _chunk,
                           (jnp.int32(0), jnp.int32(0)))

    for ps in (0, 1):
        fv, rv, gs, ss = FIRES[ps]

        @pl.when((nf >= 1) & (((nf - 1) & 1) == ps))
        def _():
            pltpu.make_async_copy(h_hbm.at[fv.at[0]], rv, gs).wait()
            _compute(fv, rv)
            pltpu.sync_copy(rv, acc.at[fv.at[1]], add=True)

    for ps in (0, 1):
        fv, rv, gs, ss = FIRES[ps]

        @pl.when((nf >= 2) & ((nf & 1) == ps))
        def _():
            pltpu.make_async_copy(rv, acc.at[fv.at[1]], ss).wait()

    for ps in (0, 1):
        fv, rv, gs, ss = FIRES[ps]

        @pl.when((wp > 0) & ((nf & 1) == ps))
        def _():
            for i in range(CH // 16):
                ds = pl.ds(i * 16, 16)
                lane = lax.iota(_i32, 16) + i * 16
                keep = lane < wp
                fv[0, ds] = jnp.where(keep, psrc[ds], 0)
                fv[1, ds] = jnp.where(keep, pdst[ds], TRASH)
                fv[2, ds] = jnp.where(keep, peid[ds], 0)
            pltpu.async_copy(h_hbm.at[fv.at[0]], rv, gs).wait()
            _compute(fv, rv)
            pltpu.sync_copy(rv, acc.at[fv.at[1]], add=True)

    plsc.subcore_barrier()
    pltpu.sync_copy(acc.at[pl.ds(sid * OSTRIPE, OSTRIPE)],
                    out_hbm.at[pl.ds(base + sid * OSTRIPE, OSTRIPE)])


@functools.lru_cache(maxsize=None)
def _sc_mesh():
    return plsc.VectorSubcoreMesh(core_axis_name="c", subcore_axis_name="s",
                                  num_cores=NC, num_subcores=NS)


@functools.lru_cache(maxsize=None)
def _aggr_kernel():
    return pl.kernel(
        _aggr_body,
        out_type=jax.ShapeDtypeStruct((NPAD, HID), _f32),
        mesh=_sc_mesh(),
        scratch_types=[
            pltpu.VMEM((SUP, 3, CH), _i32),
            pltpu.VMEM((3, CH), _i32),
            pltpu.VMEM((3, CH), _i32),
            pltpu.VMEM((288,), _i32),
            pltpu.VMEM((288,), _i32),
            pltpu.VMEM((288,), _i32),
            pltpu.VMEM((CH, HID), _f32),
            pltpu.VMEM((CH, HID), _f32),
            pltpu.VMEM((NUM_ET, HID), _f32),
            pltpu.VMEM_SHARED((ACC_ROWS, HID), _f32),
            pltpu.SemaphoreType.DMA,
            pltpu.SemaphoreType.DMA,
            pltpu.SemaphoreType.DMA,
            pltpu.SemaphoreType.DMA,
            pltpu.SemaphoreType.DMA,
        ],
        compiler_params=pltpu.CompilerParams(use_tc_tiling_on_sc=False,
                                             needs_layout_passes=False),
    )


def _aggr_call(h, idx, tab):
    return _aggr_kernel()(h, idx, tab)


def _text_body(tok_hbm, ids_hbm, mask_hbm, out_hbm,
               ids_v, mask_v, rows_v, res_v, sem):
    cid = lax.axis_index("c")
    sid = lax.axis_index("s")
    w = sid * NC + cid
    g0 = w * 2

    pltpu.sync_copy(ids_hbm.at[pl.ds(g0, 2)], ids_v)
    pltpu.sync_copy(mask_hbm.at[pl.ds(g0, 2)], mask_v)
    for j in range(2):
        pltpu.async_copy(tok_hbm.at[ids_v.at[j]], rows_v, sem).wait()

        def _acc(i, carry):
            a0, a1, a2, a3, ms = carry
            m16 = mask_v[j, pl.ds(i * 16, 16)]
            for lane in range(16):
                m = m16[lane]
                e = i * 16 + lane
                a0 = a0 + rows_v[e, pl.ds(0, 16)] * m
                a1 = a1 + rows_v[e, pl.ds(16, 16)] * m
                a2 = a2 + rows_v[e, pl.ds(32, 16)] * m
                a3 = a3 + rows_v[e, pl.ds(48, 16)] * m
            return (a0, a1, a2, a3, ms + m16)
        z = jnp.zeros((16,), _f32)
        a0, a1, a2, a3, _ = lax.fori_loop(0, L // 16, _acc,
                                          (z, z, z, z, z))
        res_v[j, pl.ds(0, 16)] = a0
        res_v[j, pl.ds(16, 16)] = a1
        res_v[j, pl.ds(32, 16)] = a2
        res_v[j, pl.ds(48, 16)] = a3
    pltpu.sync_copy(res_v, out_hbm.at[pl.ds(g0, 2)])


@functools.lru_cache(maxsize=None)
def _text_kernel():
    return pl.kernel(
        _text_body,
        out_type=jax.ShapeDtypeStruct((G, TEXT), _f32),
        mesh=_sc_mesh(),
        scratch_types=[
            pltpu.VMEM((2, L), _i32),
            pltpu.VMEM((2, L), _f32),
            pltpu.VMEM((L, TEXT), _f32),
            pltpu.VMEM((2, TEXT), _f32),
            pltpu.SemaphoreType.DMA,
        ],
        compiler_params=pltpu.CompilerParams(use_tc_tiling_on_sc=False,
                                             needs_layout_passes=False),
    )


def _text_call(tok_w, ids, mask):
    return _text_kernel()(tok_w, ids, mask)


def _embed_body(x_ref, opw_ref, ew_ref, we0, we1, we2, b0, b1, b2,
                h_ref, t0_ref, t1_ref, t2_ref):
    xb = x_ref[...]
    opi = jnp.clip(xb[:, 0:1].astype(_i32), 0, 63)
    ioh = lax.broadcasted_iota(_i32, (RBLK, 64), 1)
    oh = (opi == ioh).astype(_f32)
    emb = jnp.dot(oh, opw_ref[...], preferred_element_type=_f32)
    h_ref[...] = jnp.concatenate(
        [emb, xb[:, 1:9], jnp.zeros((RBLK, 24), _f32)], axis=1)

    @pl.when(pl.program_id(0) == 0)
    def _():
        ew = ew_ref[...]
        t0_ref[...] = jnp.dot(ew, we0[...], preferred_element_type=_f32) + b0[...]
        t1_ref[...] = jnp.dot(ew, we1[...], preferred_element_type=_f32) + b1[...]
        t2_ref[...] = jnp.dot(ew, we2[...], preferred_element_type=_f32) + b2[...]


def _mlp_body(res):
    def body(h_ref, a_ref, w1, b1, w2, b2, eps_ref, o_ref):
        h = h_ref[...]
        z = (1.0 + eps_ref[0, 0]) * h + a_ref[...]
        a = jnp.maximum(jnp.dot(z, w1[...], preferred_element_type=_f32) + b1[...], 0.0)
        hh = jnp.dot(a, w2[...], preferred_element_type=_f32) + b2[...]
        mu = jnp.mean(hh, axis=-1, keepdims=True)
        var = jnp.mean((hh - mu) ** 2, axis=-1, keepdims=True)
        ln = (hh - mu) / jnp.sqrt(var + 1e-5)
        y = ln + h if res else ln
        o_ref[...] = jnp.where(y >= 0, y, 0.1 * y)
    return body


def _pool_body(h_ref, x_ref, text_ref, mask_ref, wm1, bm1, wm2, bm2,
               o_ref, accA, accB):
    i = pl.program_id(0)

    @pl.when(i == 0)
    def _():
        accA[...] = jnp.zeros_like(accA)
        accB[...] = jnp.zeros_like(accB)

    xb = x_ref[...]
    bfl = xb[:, 9:10].astype(_i32)
    ioh = lax.broadcasted_iota(_i32, (RBLK, G), 1)
    oh = (bfl == ioh).astype(_f32)
    accA[...] += lax.dot_general(oh, h_ref[...], (((0,), (0,)), ((), ())),
                                 preferred_element_type=_f32)
    cols = jnp.concatenate(
        [jnp.ones((RBLK, 1), _f32), xb[:, 5:6], xb[:, 4:5],
         jnp.zeros((RBLK, 5), _f32)], axis=1)
    accB[...] += lax.dot_general(oh, cols, (((0,), (0,)), ((), ())),
                                 preferred_element_type=_f32)

    @pl.when(i == NBLK - 1)
    def _():
        B = accB[...]
        cnt = B[:, 0:1]
        safe = jnp.where(cnt > 0, cnt, 1.0)
        lengths = jnp.maximum(jnp.sum(mask_ref[...], axis=1, keepdims=True), 1.0)
        cat = jnp.concatenate(
            [accA[...], cnt, B[:, 1:2] / safe, B[:, 2:3] / safe,
             text_ref[...] / lengths, jnp.zeros((G, 5), _f32)], axis=1)
        hid = jnp.dot(cat, wm1[...], preferred_element_type=_f32) + bm1[...]
        hid = jnp.where(hid >= 0, hid, 0.1 * hid)
        o_ref[...] = jnp.dot(hid, wm2[...], preferred_element_type=_f32) + bm2[...]


def kernel(x, edge_index, edge_attr, batch, sql_ids, sql_mask, params):
    xp = jnp.zeros((NPAD, 128), _f32)
    xp = xp.at[:N, :9].set(x)
    xp = xp.at[:N, 9].set(batch.astype(_f32))
    xp = xp.at[N:, 9].set(127.0)

    pad_e = EPAD - E
    srcp = jnp.concatenate([edge_index[0].astype(_i32),
                            jnp.zeros((pad_e,), _i32)])
    dstp = jnp.concatenate([edge_index[1].astype(_i32),
                            jnp.full((pad_e,), NPAD, _i32)])
    eidp = jnp.concatenate([edge_attr.astype(_i32), jnp.zeros((pad_e,), _i32)])
    idx = jnp.stack([srcp, dstp, eidp], 0)
    idx = idx.reshape(3, NS, NCHK, CH).transpose(1, 2, 0, 3)
    idx = idx.reshape(NS, NCHK // SUP, SUP, 3, CH)

    convs = params['convs']
    ins = [40, 64, 64]
    wes, bes, w1s, b1s, w2s, b2s, epss = [], [], [], [], [], [], []
    for l, c in enumerate(convs):
        wes.append(jnp.zeros((16, HID), _f32).at[:, :ins[l]].set(c['We']))
        bes.append(jnp.zeros((1, HID), _f32).at[0, :ins[l]].set(c['be']))
        w1s.append(jnp.zeros((HID, HID), _f32).at[:ins[l], :].set(c['W1']))
        b1s.append(c['b1'].reshape(1, HID))
        w2s.append(c['W2'])
        b2s.append(c['b2'].reshape(1, HID))
        epss.append(jnp.asarray(c['eps'], _f32).reshape(1, 1))

    wm1 = jnp.zeros((136, HID), _f32).at[:131, :].set(params['Wm1'])
    bm1 = params['bm1'].reshape(1, HID)
    wm2 = params['Wm2']
    bm2 = params['bm2'].reshape(1, OUT)

    grid = (NBLK,)
    full = lambda s: pl.BlockSpec(s, lambda i: (0, 0))
    h0, t0, t1, t2 = pl.pallas_call(
        _embed_body,
        grid=grid,
        in_specs=[pl.BlockSpec((RBLK, 128), lambda i: (i, 0)),
                  full((64, 32)), full((NUM_ET, 16)),
                  full((16, HID)), full((16, HID)), full((16, HID)),
                  full((1, HID)), full((1, HID)), full((1, HID))],
        out_specs=[pl.BlockSpec((RBLK, HID), lambda i: (i, 0)),
                   full((NUM_ET, HID)), full((NUM_ET, HID)), full((NUM_ET, HID))],
        out_shape=[jax.ShapeDtypeStruct((NPAD, HID), _f32)] +
                  [jax.ShapeDtypeStruct((NUM_ET, HID), _f32)] * 3,
    )(xp, params['op_w'], params['edge_w'],
      wes[0], wes[1], wes[2], bes[0], bes[1], bes[2])

    tabs = [t0, t1, t2]
    h = h0
    for l in range(3):
        aggr = _aggr_call(h, idx, tabs[l])
        h = pl.pallas_call(
            _mlp_body(l > 0),
            grid=grid,
            in_specs=[pl.BlockSpec((RBLK, HID), lambda i: (i, 0)),
                      pl.BlockSpec((RBLK, HID), lambda i: (i, 0)),
                      full((HID, HID)), full((1, HID)),
                      full((HID, HID)), full((1, HID)),
                      full((1, 1))],
            out_specs=pl.BlockSpec((RBLK, HID), lambda i: (i, 0)),
            out_shape=jax.ShapeDtypeStruct((NPAD, HID), _f32),
        )(h, aggr, w1s[l], b1s[l], w2s[l], b2s[l], epss[l])

    text = _text_call(params['tok_w'], sql_ids.astype(_i32), sql_mask)

    out = pl.pallas_call(
        _pool_body,
        grid=grid,
        in_specs=[pl.BlockSpec((RBLK, HID), lambda i: (i, 0)),
                  pl.BlockSpec((RBLK, 128), lambda i: (i, 0)),
                  full((G, TEXT)), full((G, L)), full((136, HID)),
                  full((1, HID)), full((HID, OUT)), full((1, OUT))],
        out_specs=full((G, OUT)),
        out_shape=jax.ShapeDtypeStruct((G, OUT), _f32),
        scratch_shapes=[pltpu.VMEM((G, HID), _f32), pltpu.VMEM((G, 8), _f32)],
    )(h, xp, text, sql_mask.astype(_f32), wm1, bm1, wm2, bm2)
    return out
